# baseline (speedup 1.0000x reference)
import functools
from functools import partial

import jax
import jax.numpy as jnp
from jax.experimental import pallas as pl
from jax.experimental.pallas import tpu as pltpu

_E1, _E2, _E3, _E4 = 0.0096, 0.0192, 0.0384, 0.0768
_P = 16
_F32 = jnp.float32


def _lrelu(x):
    return jnp.where(x >= 0, x, 0.1 * x)


def _rt(x):
    return x.astype(jnp.bfloat16).astype(_F32)


def _kpconv_body(rel_ref, nf_ref, kp_ref, w_ref, s_ref, b_ref, y_ref, st_ref,
                 *, K, C, cout, extent, act, nvalid, blk):
    i = pl.program_id(0)
    wf = jnp.zeros((blk, _P, C), _F32)
    for k in range(K):
        relk = rel_ref[k]
        d2 = jnp.zeros((blk, _P), _F32)
        for d in range(3):
            diff = relk[:, d:d + 1] - kp_ref[d:d + 1, :]
            d2 = d2 + diff * diff
        dist = jnp.sqrt(d2 + 1e-12)
        infl = jnp.maximum(0.0, 1.0 - dist / extent)
        nfk = nf_ref[k] * s_ref[0:1, :] + b_ref[0:1, :]
        if act:
            nfk = _lrelu(nfk)
        infl = _rt(infl)
        nfk = _rt(nfk)
        wf = wf + infl[:, :, None] * nfk[:, None, :]
    y = jnp.zeros((blk, cout), _F32)
    for p_ in range(_P):
        y = y + jnp.dot(wf[:, p_, :].astype(jnp.bfloat16),
                        w_ref[p_].astype(jnp.bfloat16),
                        preferred_element_type=_F32)
    if nvalid is not None:
        rowmask = jax.lax.broadcasted_iota(jnp.int32, (blk, 1), 0) < nvalid
        y = jnp.where(rowmask, y, 0.0)
    y_ref[...] = y
    st = jnp.concatenate([jnp.sum(y, 0, keepdims=True),
                          jnp.sum(y * y, 0, keepdims=True),
                          jnp.zeros((6, cout), _F32)], axis=0)

    @pl.when(i == 0)
    def _():
        st_ref[...] = st

    @pl.when(i != 0)
    def _():
        st_ref[...] = st_ref[...] + st


def _kpconv(rel_t, nf_t, kp_t, w3, scale, bias, extent, act, blk, nvalid=None):
    K, ntot, _ = rel_t.shape
    C = nf_t.shape[2]
    cout = w3.shape[2]
    grid = ntot // blk
    body = partial(_kpconv_body, K=K, C=C, cout=cout, extent=extent,
                   act=act, nvalid=nvalid, blk=blk)
    return pl.pallas_call(
        body,
        grid=(grid,),
        in_specs=[
            pl.BlockSpec((K, blk, 3), lambda i: (0, i, 0)),
            pl.BlockSpec((K, blk, C), lambda i: (0, i, 0)),
            pl.BlockSpec((8, _P), lambda i: (0, 0)),
            pl.BlockSpec((_P, C, cout), lambda i: (0, 0, 0)),
            pl.BlockSpec((8, C), lambda i: (0, 0)),
            pl.BlockSpec((8, C), lambda i: (0, 0)),
        ],
        out_specs=[
            pl.BlockSpec((blk, cout), lambda i: (i, 0)),
            pl.BlockSpec((8, cout), lambda i: (0, 0)),
        ],
        out_shape=[
            jax.ShapeDtypeStruct((ntot, cout), _F32),
            jax.ShapeDtypeStruct((8, cout), _F32),
        ],
        compiler_params=pltpu.CompilerParams(
            dimension_semantics=("arbitrary",)),
    )(rel_t, nf_t, kp_t, w3, scale, bias)


def _mm_body(x_ref, w_ref, s_ref, b_ref, y_ref, st_ref,
             *, act, nvalid, blk, cout):
    i = pl.program_id(0)
    x = x_ref[...] * s_ref[0:1, :] + b_ref[0:1, :]
    if act:
        x = _lrelu(x)
    y = jnp.dot(x.astype(jnp.bfloat16), w_ref[...].astype(jnp.bfloat16),
                preferred_element_type=_F32)
    if nvalid is not None:
        rowmask = jax.lax.broadcasted_iota(jnp.int32, (blk, 1), 0) < nvalid
        y = jnp.where(rowmask, y, 0.0)
    y_ref[...] = y
    st = jnp.concatenate([jnp.sum(y, 0, keepdims=True),
                          jnp.sum(y * y, 0, keepdims=True),
                          jnp.zeros((6, cout), _F32)], axis=0)

    @pl.when(i == 0)
    def _():
        st_ref[...] = st

    @pl.when(i != 0)
    def _():
        st_ref[...] = st_ref[...] + st


def _mm(x, w, scale, bias, act, blk, nvalid=None):
    ntot, cin = x.shape
    cout = w.shape[1]
    grid = ntot // blk
    body = partial(_mm_body, act=act, nvalid=nvalid, blk=blk, cout=cout)
    return pl.pallas_call(
        body,
        grid=(grid,),
        in_specs=[
            pl.BlockSpec((blk, cin), lambda i: (i, 0)),
            pl.BlockSpec((cin, cout), lambda i: (0, 0)),
            pl.BlockSpec((8, cin), lambda i: (0, 0)),
            pl.BlockSpec((8, cin), lambda i: (0, 0)),
        ],
        out_specs=[
            pl.BlockSpec((blk, cout), lambda i: (i, 0)),
            pl.BlockSpec((8, cout), lambda i: (0, 0)),
        ],
        out_shape=[
            jax.ShapeDtypeStruct((ntot, cout), _F32),
            jax.ShapeDtypeStruct((8, cout), _F32),
        ],
        compiler_params=pltpu.CompilerParams(
            dimension_semantics=("arbitrary",)),
    )(x, w, scale, bias)


def _pool_body(xu_ref, xr_ref, su_ref, bu_ref, sr_ref, br_ref, pts_ref,
               ids_ref, fp_ref, pp_ref, *, mp, blk):
    i = pl.program_id(0)
    x = _lrelu(xu_ref[...] * su_ref[0:1, :] + bu_ref[0:1, :])
    r = xr_ref[...] * sr_ref[0:1, :] + br_ref[0:1, :]
    f = _lrelu(x + r)
    ids = ids_ref[0]
    oh = (jax.lax.broadcasted_iota(jnp.int32, (mp, blk), 0)
          == ids).astype(_F32)
    fp = jnp.dot(oh, f, preferred_element_type=_F32, precision=jax.lax.Precision.HIGHEST)
    pp = jnp.dot(oh, pts_ref[...], preferred_element_type=_F32, precision=jax.lax.Precision.HIGHEST)

    @pl.when(i == 0)
    def _():
        fp_ref[...] = fp
        pp_ref[...] = pp

    @pl.when(i != 0)
    def _():
        fp_ref[...] = fp_ref[...] + fp
        pp_ref[...] = pp_ref[...] + pp


def _pool(xu, xr, su, bu, sr, br, pts4, ids3, mp, blk):
    ntot, c = xu.shape
    grid = ntot // blk
    body = partial(_pool_body, mp=mp, blk=blk)
    return pl.pallas_call(
        body,
        grid=(grid,),
        in_specs=[
            pl.BlockSpec((blk, c), lambda i: (i, 0)),
            pl.BlockSpec((blk, c), lambda i: (i, 0)),
            pl.BlockSpec((8, c), lambda i: (0, 0)),
            pl.BlockSpec((8, c), lambda i: (0, 0)),
            pl.BlockSpec((8, c), lambda i: (0, 0)),
            pl.BlockSpec((8, c), lambda i: (0, 0)),
            pl.BlockSpec((blk, 4), lambda i: (i, 0)),
            pl.BlockSpec((1, 1, blk), lambda i: (i, 0, 0)),
        ],
        out_specs=[
            pl.BlockSpec((mp, c), lambda i: (0, 0)),
            pl.BlockSpec((mp, 4), lambda i: (0, 0)),
        ],
        out_shape=[
            jax.ShapeDtypeStruct((mp, c), _F32),
            jax.ShapeDtypeStruct((mp, 4), _F32),
        ],
        compiler_params=pltpu.CompilerParams(
            dimension_semantics=("arbitrary",)),
    )(xu, xr, su, bu, sr, br, pts4, ids3)


def _comb_body(xu_ref, xr_ref, su_ref, bu_ref, sr_ref, br_ref, y_ref):
    x = _lrelu(xu_ref[...] * su_ref[0:1, :] + bu_ref[0:1, :])
    r = xr_ref[...] * sr_ref[0:1, :] + br_ref[0:1, :]
    y_ref[...] = _lrelu(x + r)


def _comb(xu, xr, su, bu, sr, br):
    mp, c = xu.shape
    return pl.pallas_call(
        _comb_body,
        grid=(1,),
        in_specs=[
            pl.BlockSpec((mp, c), lambda i: (0, 0)),
            pl.BlockSpec((mp, c), lambda i: (0, 0)),
            pl.BlockSpec((8, c), lambda i: (0, 0)),
            pl.BlockSpec((8, c), lambda i: (0, 0)),
            pl.BlockSpec((8, c), lambda i: (0, 0)),
            pl.BlockSpec((8, c), lambda i: (0, 0)),
        ],
        out_specs=pl.BlockSpec((mp, c), lambda i: (0, 0)),
        out_shape=jax.ShapeDtypeStruct((mp, c), _F32),
    )(xu, xr, su, bu, sr, br)


def _head_body(xu_ref, xr_ref, su_ref, bu_ref, sr_ref, br_ref, bid_ref,
               inv_ref, w_ref, ob_ref, o_ref, *, mp):
    x = _lrelu(xu_ref[...] * su_ref[0:1, :] + bu_ref[0:1, :])
    r = xr_ref[...] * sr_ref[0:1, :] + br_ref[0:1, :]
    f = _lrelu(x + r)
    bid = bid_ref[0]
    oh = (jax.lax.broadcasted_iota(jnp.int32, (8, mp), 0)
          == bid).astype(_F32)
    g = jnp.dot(oh, f, preferred_element_type=_F32,
                precision=jax.lax.Precision.HIGHEST) * inv_ref[:, 0:1]
    g = _lrelu(g)
    o_ref[...] = jnp.dot(g.astype(jnp.bfloat16),
                         w_ref[...].astype(jnp.bfloat16),
                         preferred_element_type=_F32) + ob_ref[...]


def _head(xu, xr, su, bu, sr, br, bid3, inv8, w, ob):
    mp, c = xu.shape
    cout = w.shape[1]
    body = partial(_head_body, mp=mp)
    return pl.pallas_call(
        body,
        grid=(1,),
        in_specs=[
            pl.BlockSpec((mp, c), lambda i: (0, 0)),
            pl.BlockSpec((mp, c), lambda i: (0, 0)),
            pl.BlockSpec((8, c), lambda i: (0, 0)),
            pl.BlockSpec((8, c), lambda i: (0, 0)),
            pl.BlockSpec((8, c), lambda i: (0, 0)),
            pl.BlockSpec((8, c), lambda i: (0, 0)),
            pl.BlockSpec((1, 1, mp), lambda i: (0, 0, 0)),
            pl.BlockSpec((8, 8), lambda i: (0, 0)),
            pl.BlockSpec((c, cout), lambda i: (0, 0)),
            pl.BlockSpec((8, cout), lambda i: (0, 0)),
        ],
        out_specs=pl.BlockSpec((8, cout), lambda i: (0, 0)),
        out_shape=jax.ShapeDtypeStruct((8, cout), _F32),
    )(xu, xr, su, bu, sr, br, bid3, inv8, w, ob)


def _prep_kp(kp):
    kpt = jnp.zeros((8, _P), _F32)
    kpt = kpt.at[:3, :15].set(kp.T)
    kpt = kpt.at[:3, 15:].set(1e9)
    return kpt


def _prep_w(w, cpad):
    nkp, ci, co = w.shape
    wp = jnp.zeros((_P, cpad, co), _F32)
    wp = wp.at[:nkp, :ci, :].set(w)
    return wp


def _affine_from_stats(st, n, gamma, beta):
    mean = st[0] / n
    var = st[1] / n - mean * mean
    rstd = jax.lax.rsqrt(jnp.maximum(var, 0.0) + 1e-5)
    scale = gamma * rstd
    bias = beta - mean * scale
    c = scale.shape[0]
    return (jnp.broadcast_to(scale[None, :], (8, c)),
            jnp.broadcast_to(bias[None, :], (8, c)))


def _ident(c):
    return (jnp.ones((8, c), _F32), jnp.zeros((8, c), _F32))


def kernel(points, feats, length, neigh1, neigh2, pool_ids, batch_ids1,
           counts1, params):
    N = points.shape[0]
    M, K2 = neigh2.shape
    B = counts1.shape[0]
    p = params

    pts_pad = jnp.concatenate([points, jnp.full((1, 3), 1e6, _F32)], 0)
    n1t = neigh1.T
    n1c = jnp.clip(n1t, 0, N - 1)
    rel1 = pts_pad[n1t] - points[None, :, :]

    f8 = jnp.pad(feats, ((0, 0), (0, 5)))
    nf1 = f8[n1c]
    s_id8, b_id8 = _ident(8)
    y1, st1 = _kpconv(rel1, nf1, _prep_kp(p['b1']['kp']),
                      _prep_w(p['b1']['W'], 8), s_id8, b_id8,
                      _E1, act=False, blk=256)
    s1, b1a = _affine_from_stats(st1, N, p['b1']['bn_g'], p['b1']['bn_b'])

    b2 = p['b2']
    z2d, st2d = _mm(y1, b2['down_W'], s1, b1a, act=True, blk=512)
    s2d, b2d = _affine_from_stats(st2d, N, b2['down_g'], b2['down_b'])
    nf2 = z2d[n1c]
    y2, st2 = _kpconv(rel1, nf2, _prep_kp(b2['kp']), _prep_w(b2['W'], 32),
                      s2d, b2d, _E2, act=True, blk=256)
    s2, b2a = _affine_from_stats(st2, N, b2['bn_g'], b2['bn_b'])
    z2u, st2u = _mm(y2, b2['up_W'], s2, b2a, act=True, blk=512)
    s2u, b2u = _affine_from_stats(st2u, N, b2['up_g'], b2['up_b'])
    z2r, st2r = _mm(y1, b2['res_W'], s1, b1a, act=True, blk=512)
    s2r, b2r = _affine_from_stats(st2r, N, b2['res_g'], b2['res_b'])

    mp = max(8, -(-M // 8) * 8)
    blkp = 512
    ids3 = pool_ids.reshape(N // blkp, 1, blkp)
    pts4 = jnp.pad(points, ((0, 0), (0, 1)), constant_values=1.0)
    fpool, ppool = _pool(z2u, z2r, s2u, b2u, s2r, b2r, pts4, ids3, mp, blkp)
    cnt = jnp.maximum(ppool[:M, 3:4], 1.0)
    p1 = ppool[:M, :3] / cnt
    f1 = fpool[:M] / cnt

    pts1_pad = jnp.concatenate([p1, jnp.full((1, 3), 1e6, _F32)], 0)
    n2t = neigh2.T
    n2c = jnp.clip(n2t, 0, M - 1)
    rel2 = pts1_pad[n2t] - p1[None, :, :]
    rel2 = jnp.pad(rel2, ((0, 0), (0, mp - M), (0, 0)),
                   constant_values=1e6)
    b3 = p['b3']
    nf3 = jnp.pad(f1[n2c], ((0, 0), (0, mp - M), (0, 0)))
    s_id128, b_id128 = _ident(128)
    y3, st3 = _kpconv(rel2, nf3, _prep_kp(b3['kp']), _prep_w(b3['W'], 128),
                      s_id128, b_id128, _E3, act=False, blk=mp, nvalid=M)
    s3, b3a = _affine_from_stats(st3, M, b3['bn_g'], b3['bn_b'])
    z3u, st3u = _mm(y3, b3['up_W'], s3, b3a, act=True, blk=mp, nvalid=M)
    s3u, b3u = _affine_from_stats(st3u, M, b3['up_g'], b3['up_b'])
    f1p = jnp.pad(f1, ((0, mp - M), (0, 0)))
    z3r, st3r = _mm(f1p, b3['res_W'], s_id128, b_id128, act=False,
                    blk=mp, nvalid=M)
    s3r, b3r = _affine_from_stats(st3r, M, b3['res_g'], b3['res_b'])
    f2 = _comb(z3u, z3r, s3u, b3u, s3r, b3r)

    b4 = p['b4']
    s_id512, b_id512 = _ident(512)
    z4d, st4d = _mm(f2, b4['down_W'], s_id512, b_id512, act=False,
                    blk=mp, nvalid=M)
    s4d, b4d = _affine_from_stats(st4d, M, b4['down_g'], b4['down_b'])
    nf4 = jnp.pad(z4d[n2c], ((0, 0), (0, mp - M), (0, 0)))
    y4, st4 = _kpconv(rel2, nf4, _prep_kp(b4['kp']), _prep_w(b4['W'], 256),
                      s4d, b4d, _E4, act=True, blk=mp, nvalid=M)
    s4, b4a = _affine_from_stats(st4, M, b4['bn_g'], b4['bn_b'])
    z4u, st4u = _mm(y4, b4['up_W'], s4, b4a, act=True, blk=mp, nvalid=M)
    s4u, b4u = _affine_from_stats(st4u, M, b4['up_g'], b4['up_b'])
    z4r, st4r = _mm(f2, b4['res_W'], s_id512, b_id512, act=False,
                    blk=mp, nvalid=M)
    s4r, b4r = _affine_from_stats(st4r, M, b4['res_g'], b4['res_b'])

    bid = jnp.pad(batch_ids1, (0, mp - M), constant_values=-1)
    bid3 = bid.reshape(1, 1, mp)
    inv = 1.0 / jnp.maximum(counts1, 1).astype(_F32)
    inv8 = jnp.broadcast_to(jnp.pad(inv, (0, 8 - B),
                                    constant_values=1.0)[:, None], (8, 8))
    ob = jnp.broadcast_to(p['out_b'][None, :], (8, p['out_b'].shape[0]))
    out = _head(z4u, z4r, s4u, b4u, s4r, b4r, bid3, inv8, p['out_W'], ob)
    return out[:B]
